# Initial kernel scaffold; baseline (speedup 1.0000x reference)
import jax
import jax.numpy as jnp
from jax.experimental import pallas as pl


def kernel(output, targets):
    raise NotImplementedError("write your pallas kernel here")


# speedup vs baseline: 1.8012x; 1.8012x over previous
import functools
import jax
import jax.numpy as jnp
from jax import lax
from jax.experimental import pallas as pl
from jax.experimental.pallas import tpu as pltpu

_ANCHOR_R = (2.5, 5.0, 10.0)
_ANCHOR_V = (125.0, 1000.0, 8000.0)
_TH_POS = 0.5
_TH_NEG = 0.02
_NUM_NEG = 800
_POS_W = 5.0
_REG_W = 0.5

_B = 4
_N = 32 * 32 * 32 * 3
_ROWS = _N // 128


def _bce_pos(x):
    return jnp.maximum(x, 0.0) - x + jnp.log1p(jnp.exp(-jnp.abs(x)))


def _bce_neg(x):
    return jnp.maximum(x, 0.0) + jnp.log1p(jnp.exp(-jnp.abs(x)))


def _smooth_l1(d):
    a = jnp.abs(d)
    return jnp.where(a < 1.0, 0.5 * a * a, a - 0.5)


def _mono_key_u32(v):
    bi = lax.bitcast_convert_type(v, jnp.int32)
    m = lax.shift_right_arithmetic(bi, 31)
    ki = bi ^ (m | jnp.int32(-2147483648))
    return lax.bitcast_convert_type(ki, jnp.uint32)


def _unmono_f32(k):
    ki = lax.bitcast_convert_type(k, jnp.int32)
    neg = ki >= 0
    bi = jnp.where(neg, ~ki, ki ^ jnp.int32(-2147483648))
    return lax.bitcast_convert_type(bi, jnp.float32)


def _loss_kernel(out_ref, tgt_ref, total_ref, cls_ref, reg_ref, npos_ref):
    r = lax.broadcasted_iota(jnp.int32, (_ROWS, 128), 0)
    l = lax.broadcasted_iota(jnp.int32, (_ROWS, 128), 1)
    idx = r * 128 + l
    a = idx % 3
    x = (idx // 3) % 32
    y = (idx // 96) % 32
    z = idx // 3072
    xf = x.astype(jnp.float32) * 4.0 + 2.0
    yf = y.astype(jnp.float32) * 4.0 + 2.0
    zf = z.astype(jnp.float32) * 4.0 + 2.0
    a0 = a == 0
    a1 = a == 1
    ra = jnp.where(a0, _ANCHOR_R[0], jnp.where(a1, _ANCHOR_R[1], _ANCHOR_R[2]))
    va = jnp.where(a0, _ANCHOR_V[0], jnp.where(a1, _ANCHOR_V[1], _ANCHOR_V[2]))
    asz = 2.0 * ra

    tcls = jnp.float32(0.0)
    treg_l = jnp.float32(0.0)
    tpos = jnp.float32(0.0)

    for b in range(_B):
        conf = out_ref[b, 0]
        best_iou = jnp.full((_ROWS, 128), -1.0, jnp.float32)
        mgx = jnp.zeros((_ROWS, 128), jnp.float32)
        mgy = jnp.zeros((_ROWS, 128), jnp.float32)
        mgz = jnp.zeros((_ROWS, 128), jnp.float32)
        mgd = jnp.zeros((_ROWS, 128), jnp.float32)
        for g in range(8):
            gx = tgt_ref[b, g, 0]
            gy = tgt_ref[b, g, 1]
            gz = tgt_ref[b, g, 2]
            gd = tgt_ref[b, g, 3]
            h = gd * 0.5
            ox = jnp.maximum(0.0, jnp.minimum(xf + ra, gx + h) - jnp.maximum(xf - ra, gx - h))
            oy = jnp.maximum(0.0, jnp.minimum(yf + ra, gy + h) - jnp.maximum(yf - ra, gy - h))
            oz = jnp.maximum(0.0, jnp.minimum(zf + ra, gz + h) - jnp.maximum(zf - ra, gz - h))
            inter = ox * oy * oz
            den = va + gd * gd * gd - inter + 1e-6
            iou = inter / den
            upd = iou > best_iou
            best_iou = jnp.where(upd, iou, best_iou)
            mgx = jnp.where(upd, gx, mgx)
            mgy = jnp.where(upd, gy, mgy)
            mgz = jnp.where(upd, gz, mgz)
            mgd = jnp.where(upd, gd, mgd)

        pos = best_iou > _TH_POS
        neg = best_iou < _TH_NEG
        pos_f = pos.astype(jnp.float32)
        npos = jnp.sum(pos_f)
        nneg = jnp.sum(neg.astype(jnp.float32))

        pos_sum = jnp.sum(jnp.where(pos, _bce_pos(conf), 0.0))
        pos_loss = jnp.where(npos > 0, _POS_W * pos_sum / jnp.maximum(npos, 1.0), 0.0)

        neg_vals = jnp.where(neg, _bce_neg(conf), -1e9)
        keys = _mono_key_u32(neg_vals)
        kk = jnp.minimum(nneg, jnp.float32(_NUM_NEG))
        kkc = jnp.maximum(kk, 1.0)
        kkc_i = kkc.astype(jnp.int32)

        def bs_body(_, lohi):
            lo, hi = lohi
            mid = lo + lax.shift_right_logical(hi - lo, jnp.uint32(1))
            cnt = jnp.sum((keys > mid).astype(jnp.int32))
            smaller = cnt < kkc_i
            return (jnp.where(smaller, lo, mid + jnp.uint32(1)),
                    jnp.where(smaller, mid, hi))

        lo0 = jnp.uint32(0)
        hi0 = jnp.uint32(0xFFFFFFFF)
        lo, hi = lax.fori_loop(0, 32, bs_body, (lo0, hi0))
        thr_key = lo
        thr_val = _unmono_f32(thr_key)
        above = keys > thr_key
        n1 = jnp.sum(above.astype(jnp.float32))
        s1 = jnp.sum(jnp.where(above, neg_vals, 0.0))
        top_sum = s1 + (kkc - n1) * thr_val
        neg_loss = jnp.where(nneg > 0, top_sum / kkc, 0.0)

        cls_loss = pos_loss + neg_loss

        reg_sum = jnp.float32(0.0)
        for c, (mgc, cf) in enumerate(((mgx, xf), (mgy, yf), (mgz, zf))):
            d = (mgc - cf) / asz
            reg_sum += jnp.sum(jnp.where(pos, _smooth_l1(out_ref[b, 1 + c] - d), 0.0))
        dd = jnp.log(mgd / asz)
        reg_sum += jnp.sum(jnp.where(pos, _smooth_l1(out_ref[b, 4] - dd), 0.0))
        reg_loss = jnp.where(npos > 0, reg_sum / jnp.maximum(npos * 4.0, 1.0), 0.0)

        tcls += cls_loss
        treg_l += reg_loss
        tpos += npos

    tcls = tcls / _B
    treg_l = treg_l / _B
    total_ref[0, 0] = tcls + _REG_W * treg_l
    cls_ref[0, 0] = tcls
    reg_ref[0, 0] = treg_l
    npos_ref[0, 0] = tpos


@jax.jit
def kernel(output, targets):
    out_t = jnp.moveaxis(output.reshape(_B, _N, 5), 2, 1).reshape(_B, 5, _ROWS, 128)
    scal = pl.pallas_call(
        _loss_kernel,
        out_shape=[jax.ShapeDtypeStruct((1, 1), jnp.float32)] * 4,
        out_specs=[pl.BlockSpec(memory_space=pltpu.SMEM)] * 4,
        in_specs=[
            pl.BlockSpec(memory_space=pltpu.VMEM),
            pl.BlockSpec(memory_space=pltpu.SMEM),
        ],
    )(out_t, targets)
    total, cls_l, reg_l, npos = [s.reshape(()) for s in scal]
    return (total, cls_l, reg_l, npos)
